# Initial kernel scaffold; baseline (speedup 1.0000x reference)
import jax
import jax.numpy as jnp
from jax.experimental import pallas as pl


def kernel(x, edge_index, edge_weights, W, b):
    raise NotImplementedError("write your pallas kernel here")


# speedup vs baseline: 24.2715x; 24.2715x over previous
import functools

import jax
import jax.numpy as jnp
from jax import lax
from jax.experimental import pallas as pl
from jax.experimental.pallas import tpu as pltpu
from jax.experimental.pallas import tpu_sc as plsc

NC, NS, LANES = 2, 16, 16
NW = NC * NS
KW = 80
SCROWS = 25

N = 10000
E = 320000
D = 128
BN = 1000
NPT = N // NS
RPW = (E // KW) // NW


def _deg_body(dst2, ew2, degp, idx_d, ew_v, zb, acc1, sem_sc):
    c = lax.axis_index("c")
    s = lax.axis_index("s")
    wid = c * NS + s
    @pl.when(s < 5)
    def _():
        @pl.loop(0, zb.shape[0] // LANES)
        def _(i):
            zb[pl.ds(i * LANES, LANES)] = jnp.zeros((LANES,), jnp.float32)

        pltpu.sync_copy(zb, acc1.at[pl.ds(s * 2000, 2000)])

    plsc.subcore_barrier()

    @pl.loop(0, RPW // SCROWS)
    def _(ci):
        blk = wid * (RPW // SCROWS) + ci
        pltpu.sync_copy(dst2.at[blk], idx_d)
        pltpu.sync_copy(ew2.at[blk], ew_v)
        descs = [
            pltpu.async_copy(ew_v.at[j], acc1.at[idx_d.at[j]], sem_sc, add=True)
            for j in range(SCROWS)
        ]
        for dsc in descs:
            dsc.wait()

    plsc.subcore_barrier()

    @pl.when(s == 0)
    def _():
        pltpu.sync_copy(acc1, degp.at[c, 0])


_deg_call = functools.partial(
    pl.kernel,
    out_type=jax.ShapeDtypeStruct((NC, 1, N), jnp.float32),
    mesh=plsc.VectorSubcoreMesh(
        core_axis_name="c", subcore_axis_name="s", num_cores=NC, num_subcores=NS
    ),
    scratch_types=[
        pltpu.VMEM((SCROWS, KW), jnp.int32),
        pltpu.VMEM((SCROWS, KW), jnp.float32),
        pltpu.VMEM((2000,), jnp.float32),
        pltpu.VMEM_SHARED((N,), jnp.float32),
        pltpu.SemaphoreType.DMA,
    ],
    compiler_params=pltpu.CompilerParams(needs_layout_passes=False),
)(_deg_body)


def _agg_body(y, src2, dst2, ew1, outp, idx_s, idx_d, ew_v, rows0, rows1, zb,
              acc, sem_g0, sem_g1, sem_s0, sem_s1):
    c = lax.axis_index("c")
    s = lax.axis_index("s")
    wid = c * NS + s

    @pl.loop(0, zb.shape[0])
    def _(i):
        for r in range(D // LANES):
            zb[i, pl.ds(r * LANES, LANES)] = jnp.zeros((LANES,), jnp.float32)

    for k in range(NPT // zb.shape[0]):
        pltpu.sync_copy(zb, acc.at[pl.ds(s * NPT + k * zb.shape[0], zb.shape[0])])

    plsc.subcore_barrier()

    bufs = (rows0, rows1)
    gsems = (sem_g0, sem_g1)
    ssems = (sem_s0, sem_s1)

    @pl.loop(0, RPW // SCROWS)
    def _(ci):
        blk = wid * (RPW // SCROWS) + ci
        pltpu.sync_copy(src2.at[blk], idx_s)
        pltpu.sync_copy(dst2.at[blk], idx_d)
        pltpu.sync_copy(ew1.at[pl.ds(blk * (SCROWS * KW), SCROWS * KW)], ew_v)
        gd = [None] * SCROWS
        sd = [None] * SCROWS
        gd[0] = pltpu.async_copy(y.at[idx_s.at[0]], rows0, sem_g0)
        for j in range(SCROWS):
            b = j & 1
            gd[j].wait()
            if j >= 1:
                sd[j - 1].wait()
            if j + 1 < SCROWS:
                gd[j + 1] = pltpu.async_copy(
                    y.at[idx_s.at[j + 1]], bufs[(j + 1) & 1], gsems[(j + 1) & 1]
                )
            rows = bufs[b]

            @pl.loop(0, KW)
            def _(e):
                ews = plsc.load_gather(
                    ew_v, [jnp.full((LANES,), e, jnp.int32) + (j * KW)]
                )
                for r in range(D // LANES):
                    rows[e, pl.ds(r * LANES, LANES)] = (
                        rows[e, pl.ds(r * LANES, LANES)] * ews
                    )

            sd[j] = pltpu.async_copy(rows, acc.at[idx_d.at[j]], ssems[b], add=True)
        sd[SCROWS - 1].wait()

    plsc.subcore_barrier()
    pltpu.sync_copy(acc.at[pl.ds(s * NPT, NPT)], outp.at[c, s])


_agg_call = functools.partial(
    pl.kernel,
    out_type=jax.ShapeDtypeStruct((NC, NS, NPT, D), jnp.float32),
    mesh=plsc.VectorSubcoreMesh(
        core_axis_name="c", subcore_axis_name="s", num_cores=NC, num_subcores=NS
    ),
    scratch_types=[
        pltpu.VMEM((SCROWS, KW), jnp.int32),
        pltpu.VMEM((SCROWS, KW), jnp.int32),
        pltpu.VMEM((SCROWS * KW,), jnp.float32),
        pltpu.VMEM((KW, D), jnp.float32),
        pltpu.VMEM((KW, D), jnp.float32),
        pltpu.VMEM((125, D), jnp.float32),
        pltpu.VMEM_SHARED((N, D), jnp.float32),
        pltpu.SemaphoreType.DMA,
        pltpu.SemaphoreType.DMA,
        pltpu.SemaphoreType.DMA,
        pltpu.SemaphoreType.DMA,
    ],
    compiler_params=pltpu.CompilerParams(needs_layout_passes=False),
)(_agg_body)


def _dense_body(x_ref, w_ref, degp_ref, y_ref, dis_ref):
    dp = degp_ref[...]
    deg = dp[0] + dp[1]
    pos = deg > 0.0
    dis = jnp.where(pos, lax.rsqrt(jnp.where(pos, deg, 1.0)), 0.0)
    xw = jnp.dot(x_ref[...], w_ref[...], preferred_element_type=jnp.float32)
    y_ref[...] = xw * dis
    dis_ref[...] = dis


def _dense_call(x, w, degp3):
    return pl.pallas_call(
        _dense_body,
        grid=(N // BN,),
        in_specs=[
            pl.BlockSpec((BN, D), lambda g: (g, 0)),
            pl.BlockSpec((D, D), lambda g: (0, 0)),
            pl.BlockSpec((NC, BN, 1), lambda g: (0, g, 0)),
        ],
        out_specs=[
            pl.BlockSpec((BN, D), lambda g: (g, 0)),
            pl.BlockSpec((BN, 1), lambda g: (g, 0)),
        ],
        out_shape=[
            jax.ShapeDtypeStruct((N, D), jnp.float32),
            jax.ShapeDtypeStruct((N, 1), jnp.float32),
        ],
    )(x, w, degp3)


def _final_body(outp_ref, dis_ref, b_ref, o_ref):
    t = outp_ref[0] + outp_ref[1]
    o_ref[...] = jnp.maximum(t * dis_ref[...] + b_ref[...], 0.0)


def _final_call(outp, dis, b2):
    return pl.pallas_call(
        _final_body,
        grid=(N // BN,),
        in_specs=[
            pl.BlockSpec((NC, BN, D), lambda g: (0, g, 0)),
            pl.BlockSpec((BN, 1), lambda g: (g, 0)),
            pl.BlockSpec((1, D), lambda g: (0, 0)),
        ],
        out_specs=pl.BlockSpec((BN, D), lambda g: (g, 0)),
        out_shape=jax.ShapeDtypeStruct((N, D), jnp.float32),
    )(outp, dis, b2)


def kernel(x, edge_index, edge_weights, W, b):
    nblk = E // (KW * SCROWS)
    ei = edge_index.astype(jnp.int32)
    src3 = ei[0].reshape(nblk, SCROWS, KW)
    dst3 = ei[1].reshape(nblk, SCROWS, KW)
    ew3 = edge_weights.reshape(nblk, SCROWS, KW)
    degp = _deg_call(dst3, ew3)
    y, dis = _dense_call(x, W, degp.reshape(NC, N, 1))
    outp = _agg_call(y, src3, dst3, edge_weights)
    out = _final_call(outp.reshape(NC, N, D), dis, b.reshape(1, D))
    return (out, edge_index, edge_weights)


# speedup vs baseline: 24.7434x; 1.0194x over previous
import functools

import jax
import jax.numpy as jnp
from jax import lax
from jax.experimental import pallas as pl
from jax.experimental.pallas import tpu as pltpu
from jax.experimental.pallas import tpu_sc as plsc

NC, NS, LANES = 2, 16, 16
NW = NC * NS
KW = 80
SCROWS = 25

N = 10000
E = 320000
D = 128
BN = 1000
NPT = N // NS
RPW = (E // KW) // NW


def _deg_body(dst2, ew2, degp, idx_d, ew_v, zb, acc1, sem_sc):
    c = lax.axis_index("c")
    s = lax.axis_index("s")
    wid = c * NS + s
    @pl.when(s < 5)
    def _():
        @pl.loop(0, zb.shape[0] // LANES)
        def _(i):
            zb[pl.ds(i * LANES, LANES)] = jnp.zeros((LANES,), jnp.float32)

        pltpu.sync_copy(zb, acc1.at[pl.ds(s * 2000, 2000)])

    plsc.subcore_barrier()

    @pl.loop(0, RPW // SCROWS)
    def _(ci):
        blk = wid * (RPW // SCROWS) + ci
        pltpu.sync_copy(dst2.at[blk], idx_d)
        pltpu.sync_copy(ew2.at[blk], ew_v)
        descs = [
            pltpu.async_copy(ew_v.at[j], acc1.at[idx_d.at[j]], sem_sc, add=True)
            for j in range(SCROWS)
        ]
        for dsc in descs:
            dsc.wait()

    plsc.subcore_barrier()

    @pl.when(s == 0)
    def _():
        pltpu.sync_copy(acc1, degp.at[c, 0])


_deg_call = functools.partial(
    pl.kernel,
    out_type=jax.ShapeDtypeStruct((NC, 1, N), jnp.float32),
    mesh=plsc.VectorSubcoreMesh(
        core_axis_name="c", subcore_axis_name="s", num_cores=NC, num_subcores=NS
    ),
    scratch_types=[
        pltpu.VMEM((SCROWS, KW), jnp.int32),
        pltpu.VMEM((SCROWS, KW), jnp.float32),
        pltpu.VMEM((2000,), jnp.float32),
        pltpu.VMEM_SHARED((N,), jnp.float32),
        pltpu.SemaphoreType.DMA,
    ],
    compiler_params=pltpu.CompilerParams(needs_layout_passes=False),
)(_deg_body)


def _agg_body(y, src2, dst2, ew1, outp, idx_s, idx_d, ew_v, rows0, rows1, zb,
              acc, sem_g0, sem_g1, sem_s0, sem_s1):
    c = lax.axis_index("c")
    s = lax.axis_index("s")
    wid = c * NS + s

    @pl.loop(0, zb.shape[0])
    def _(i):
        for r in range(D // LANES):
            zb[i, pl.ds(r * LANES, LANES)] = jnp.zeros((LANES,), jnp.float32)

    for k in range(NPT // zb.shape[0]):
        pltpu.sync_copy(zb, acc.at[pl.ds(s * NPT + k * zb.shape[0], zb.shape[0])])

    plsc.subcore_barrier()

    bufs = (rows0, rows1)
    gsems = (sem_g0, sem_g1)
    ssems = (sem_s0, sem_s1)

    @pl.loop(0, RPW // SCROWS)
    def _(ci):
        blk = wid * (RPW // SCROWS) + ci
        pltpu.sync_copy(src2.at[blk], idx_s)
        pltpu.sync_copy(dst2.at[blk], idx_d)
        pltpu.sync_copy(ew1.at[pl.ds(blk * (SCROWS * KW), SCROWS * KW)], ew_v)
        gd = [None] * SCROWS
        sd = [None] * SCROWS
        gd[0] = pltpu.async_copy(y.at[idx_s.at[0]], rows0, sem_g0)
        for j in range(SCROWS):
            b = j & 1
            gd[j].wait()
            if j >= 1:
                sd[j - 1].wait()
            if j + 1 < SCROWS:
                gd[j + 1] = pltpu.async_copy(
                    y.at[idx_s.at[j + 1]], bufs[(j + 1) & 1], gsems[(j + 1) & 1]
                )
            rows = bufs[b]

            @pl.loop(0, KW, unroll=4)
            def _(e):
                ews = plsc.load_gather(
                    ew_v, [jnp.full((LANES,), e, jnp.int32) + (j * KW)]
                )
                for r in range(D // LANES):
                    rows[e, pl.ds(r * LANES, LANES)] = (
                        rows[e, pl.ds(r * LANES, LANES)] * ews
                    )

            sd[j] = pltpu.async_copy(rows, acc.at[idx_d.at[j]], ssems[b], add=True)
        sd[SCROWS - 1].wait()

    plsc.subcore_barrier()
    pltpu.sync_copy(acc.at[pl.ds(s * NPT, NPT)], outp.at[c, s])


_agg_call = functools.partial(
    pl.kernel,
    out_type=jax.ShapeDtypeStruct((NC, NS, NPT, D), jnp.float32),
    mesh=plsc.VectorSubcoreMesh(
        core_axis_name="c", subcore_axis_name="s", num_cores=NC, num_subcores=NS
    ),
    scratch_types=[
        pltpu.VMEM((SCROWS, KW), jnp.int32),
        pltpu.VMEM((SCROWS, KW), jnp.int32),
        pltpu.VMEM((SCROWS * KW,), jnp.float32),
        pltpu.VMEM((KW, D), jnp.float32),
        pltpu.VMEM((KW, D), jnp.float32),
        pltpu.VMEM((125, D), jnp.float32),
        pltpu.VMEM_SHARED((N, D), jnp.float32),
        pltpu.SemaphoreType.DMA,
        pltpu.SemaphoreType.DMA,
        pltpu.SemaphoreType.DMA,
        pltpu.SemaphoreType.DMA,
    ],
    compiler_params=pltpu.CompilerParams(needs_layout_passes=False),
)(_agg_body)


def _dense_body(x_ref, w_ref, degp_ref, y_ref, dis_ref):
    dp = degp_ref[...]
    deg = dp[0] + dp[1]
    pos = deg > 0.0
    dis = jnp.where(pos, lax.rsqrt(jnp.where(pos, deg, 1.0)), 0.0)
    xw = jnp.dot(x_ref[...], w_ref[...], preferred_element_type=jnp.float32)
    y_ref[...] = xw * dis
    dis_ref[...] = dis


def _dense_call(x, w, degp3):
    return pl.pallas_call(
        _dense_body,
        grid=(N // BN,),
        in_specs=[
            pl.BlockSpec((BN, D), lambda g: (g, 0)),
            pl.BlockSpec((D, D), lambda g: (0, 0)),
            pl.BlockSpec((NC, BN, 1), lambda g: (0, g, 0)),
        ],
        out_specs=[
            pl.BlockSpec((BN, D), lambda g: (g, 0)),
            pl.BlockSpec((BN, 1), lambda g: (g, 0)),
        ],
        out_shape=[
            jax.ShapeDtypeStruct((N, D), jnp.float32),
            jax.ShapeDtypeStruct((N, 1), jnp.float32),
        ],
    )(x, w, degp3)


def _final_body(outp_ref, dis_ref, b_ref, o_ref):
    t = outp_ref[0] + outp_ref[1]
    o_ref[...] = jnp.maximum(t * dis_ref[...] + b_ref[...], 0.0)


def _final_call(outp, dis, b2):
    return pl.pallas_call(
        _final_body,
        grid=(N // BN,),
        in_specs=[
            pl.BlockSpec((NC, BN, D), lambda g: (0, g, 0)),
            pl.BlockSpec((BN, 1), lambda g: (g, 0)),
            pl.BlockSpec((1, D), lambda g: (0, 0)),
        ],
        out_specs=pl.BlockSpec((BN, D), lambda g: (g, 0)),
        out_shape=jax.ShapeDtypeStruct((N, D), jnp.float32),
    )(outp, dis, b2)


def kernel(x, edge_index, edge_weights, W, b):
    nblk = E // (KW * SCROWS)
    ei = edge_index.astype(jnp.int32)
    src3 = ei[0].reshape(nblk, SCROWS, KW)
    dst3 = ei[1].reshape(nblk, SCROWS, KW)
    ew3 = edge_weights.reshape(nblk, SCROWS, KW)
    degp = _deg_call(dst3, ew3)
    y, dis = _dense_call(x, W, degp.reshape(NC, N, 1))
    outp = _agg_call(y, src3, dst3, edge_weights)
    out = _final_call(outp.reshape(NC, N, D), dis, b.reshape(1, D))
    return (out, edge_index, edge_weights)
